# baseline (speedup 1.0000x reference)
import functools

import jax
import jax.numpy as jnp
from jax import lax
from jax.experimental import pallas as pl
from jax.experimental.pallas import tpu as pltpu
from jax.experimental.pallas import tpu_sc as plsc

NC, NS, LANES = 2, 16, 16
NW = NC * NS
VOCAB, D = 100000, 64
B, L = 4096, 200
ROWS_W = B // NW
RPB = 2
NSLOT = ROWS_W // RPB
GRP = L // LANES
TAIL = L - GRP * LANES

_mesh = plsc.VectorSubcoreMesh(
    core_axis_name="c", subcore_axis_name="s", num_cores=NC, num_subcores=NS
)


def _sc_body(table_hbm, idx_hbm, ratio_hbm, out_hbm, idx_v, ratio_v,
             ra0, ra1, rb0, rb1, gs0, gs1, os0, os1):
    wid = lax.axis_index("s") * NC + lax.axis_index("c")
    row0 = wid * ROWS_W
    rows = ((ra0, ra1), (rb0, rb1))
    gsem = (gs0, gs1)
    osem = (os0, os1)

    pltpu.sync_copy(idx_hbm.at[pl.ds(row0, ROWS_W)], idx_v)
    pltpu.sync_copy(ratio_hbm.at[pl.ds(row0, ROWS_W)], ratio_v)

    def fire_gathers(s, b):
        for j in range(RPB):
            pltpu.async_copy(
                table_hbm.at[idx_v.at[s * RPB + j]],
                rows[b][j],
                gsem[b],
            )

    def wait_gathers(b):
        for j in range(RPB):
            pltpu.make_async_copy(
                table_hbm.at[idx_v.at[j]],
                rows[b][j],
                gsem[b],
            ).wait()

    def wait_outs(b):
        for j in range(RPB):
            pltpu.make_async_copy(
                rows[b][j],
                out_hbm.at[0],
                osem[b],
            ).wait()

    def slot(s, b):
        @pl.when(s >= 1)
        def _():
            wait_outs(1 - b)

        @pl.when(s + 1 < NSLOT)
        def _():
            fire_gathers(s + 1, 1 - b)

        wait_gathers(b)
        for j in range(RPB):
            r = s * RPB + j

            def mul_body(t, c):
                rv = ratio_v[r, pl.ds(t * LANES, LANES)] * 8.0
                for k in range(LANES):
                    rvec = jnp.full((LANES,), rv[k], jnp.float32)
                    tok = t * LANES + k
                    for q in range(D // LANES):
                        sl = pl.ds(q * LANES, LANES)
                        rows[b][j][tok, sl] = rows[b][j][tok, sl] * rvec
                return c

            lax.fori_loop(0, GRP, mul_body, 0)
            rvt = ratio_v[r, pl.ds(L - LANES, LANES)] * 8.0
            for k in range(LANES - TAIL, LANES):
                rvec = jnp.full((LANES,), rvt[k], jnp.float32)
                tok = L - LANES + k
                for q in range(D // LANES):
                    sl = pl.ds(q * LANES, LANES)
                    rows[b][j][tok, sl] = rows[b][j][tok, sl] * rvec
            pltpu.async_copy(rows[b][j], out_hbm.at[row0 + r], osem[b])

    fire_gathers(0, 0)

    def loop_body(t, c):
        slot(2 * t, 0)
        slot(2 * t + 1, 1)
        return c

    lax.fori_loop(0, NSLOT // 2, loop_body, 0)

    wait_outs(1)


_sc_call = functools.partial(
    pl.kernel,
    out_type=jax.ShapeDtypeStruct((B, L, D), jnp.float32),
    mesh=_mesh,
    compiler_params=pltpu.CompilerParams(use_tc_tiling_on_sc=False),
    scratch_types=[
        pltpu.VMEM((ROWS_W, L), jnp.int32),
        pltpu.VMEM((ROWS_W, L), jnp.float32),
        pltpu.VMEM((L, D), jnp.float32),
        pltpu.VMEM((L, D), jnp.float32),
        pltpu.VMEM((L, D), jnp.float32),
        pltpu.VMEM((L, D), jnp.float32),
        pltpu.SemaphoreType.DMA,
        pltpu.SemaphoreType.DMA,
        pltpu.SemaphoreType.DMA,
        pltpu.SemaphoreType.DMA,
    ],
)(_sc_body)


def kernel(x, table):
    words = x[:, 0, :].astype(jnp.int32)
    ratio = x[:, 1, :]
    return _sc_call(table, words, ratio)
